# baseline (speedup 1.0000x reference)
import dataclasses
import functools

import jax
import jax.numpy as jnp
from jax import lax
from jax.experimental import pallas as pl
from jax.experimental.pallas import tpu as pltpu
from jax.experimental.pallas import tpu_sc as plsc

_NC = 2
_NS = 16
_NW = _NC * _NS
_L = 16
_CH = 1024


def _compiler_params():
    cp = pltpu.CompilerParams()
    fields = pltpu.CompilerParams.__dataclass_fields__
    if "needs_layout_passes" in fields:
        cp = dataclasses.replace(cp, needs_layout_passes=False)
    if "use_tc_tiling_on_sc" in fields:
        cp = dataclasses.replace(cp, use_tc_tiling_on_sc=False)
    return cp


def _split_edges_tc(edge_index):
    e = edge_index.shape[1]
    blk = 1
    for cand in (256000, 51200, 10240, 2048, 1024):
        if e % cand == 0:
            blk = cand
            break
    grid = e // blk

    def body(x_ref, r_ref, c_ref):
        r_ref[...] = x_ref[0, :]
        c_ref[...] = x_ref[1, :]

    return pl.pallas_call(
        body,
        grid=(grid,),
        in_specs=[pl.BlockSpec((2, blk), lambda i: (0, i))],
        out_specs=(
            pl.BlockSpec((blk,), lambda i: (i,)),
            pl.BlockSpec((blk,), lambda i: (i,)),
        ),
        out_shape=(
            jax.ShapeDtypeStruct((e,), jnp.int32),
            jax.ShapeDtypeStruct((e,), jnp.int32),
        ),
    )(edge_index)


@functools.partial(jax.jit, static_argnames=("e", "n"))
def _sc_periodic_distance(table, row, col, cell, *, e, n):
    del n
    assert e % _CH == 0 and e % 128 == 0
    n_chunks = e // _CH
    nfull, rem = divmod(n_chunks, _NW)
    maxt = nfull + (1 if rem else 0)
    maxt += maxt % 2
    ngrp = _CH // 128

    mesh = plsc.VectorSubcoreMesh(core_axis_name="c", subcore_axis_name="s")

    @functools.partial(
        pl.kernel,
        compiler_params=_compiler_params(),
        out_type=(
            jax.ShapeDtypeStruct((e,), jnp.float32),
            jax.ShapeDtypeStruct((4 * e,), jnp.float32),
        ),
        mesh=mesh,
        scratch_types=[
            [pltpu.VMEM((_CH,), jnp.int32) for _ in range(2)],
            [pltpu.VMEM((_CH,), jnp.int32) for _ in range(2)],
            [pltpu.VMEM((_CH, 8), jnp.float32) for _ in range(2)],
            [pltpu.VMEM((_CH, 8), jnp.float32) for _ in range(2)],
            [pltpu.VMEM((_CH,), jnp.float32) for _ in range(2)],
            [pltpu.VMEM((4 * _CH,), jnp.float32) for _ in range(2)],
            pltpu.VMEM((3, 3), jnp.float32),
            pltpu.VMEM_SHARED((100000, 8), jnp.float32),
            [pltpu.SemaphoreType.DMA for _ in range(2)],
            [pltpu.SemaphoreType.DMA for _ in range(2)],
            [pltpu.SemaphoreType.DMA for _ in range(2)],
        ],
    )
    def k(table_hbm, row_hbm, col_hbm, cell_hbm, dist_hbm, dflat_hbm,
          ridx_v, cidx_v, a_v, b_v, dist_v, delta_v, cell_v, table_sh,
          isem, gsem, osem):
        wid = lax.axis_index("c") * _NS + lax.axis_index("s")
        t_w = nfull + jnp.where(wid < rem, 1, 0)
        pltpu.sync_copy(cell_hbm, cell_v)

        @pl.when(lax.axis_index("s") == 0)
        def _():
            pltpu.sync_copy(table_hbm, table_sh)

        plsc.subcore_barrier()

        lanes = lax.iota(jnp.int32, _L)
        crow = jnp.where(lanes < 9, lanes, 0) // 3
        ccol = jnp.where(lanes < 9, lanes, 0) % 3
        cv = plsc.load_gather(cell_v, [crow, ccol])
        c00 = cv[0]
        c01 = cv[1]
        c02 = cv[2]
        c10 = cv[3]
        c11 = cv[4]
        c12 = cv[5]
        c20 = cv[6]
        c21 = cv[7]
        c22 = cv[8]

        half = jnp.float32(0.5)
        one = jnp.float32(1.0)
        zeros = jnp.zeros((_L,), jnp.float32)

        for b in range(2):
            for g in range(ngrp):
                for j in range(0, 128, _L):
                    delta_v[b][pl.ds(g * 512 + 384 + j, _L)] = zeros

        def minimage(d):
            d = jnp.where(d > half, d - one, d)
            return jnp.where(d < -half, d + one, d)

        def off_of(i):
            return (wid + _NW * i) * _CH

        def issue_idx(i, b):
            off = off_of(i)
            pltpu.async_copy(row_hbm.at[pl.ds(off, _CH)], ridx_v[b], isem[b])
            pltpu.async_copy(col_hbm.at[pl.ds(off, _CH)], cidx_v[b], isem[b])

        def wait_idx(b):
            pltpu.make_async_copy(row_hbm.at[pl.ds(0, _CH)], ridx_v[b],
                                  isem[b]).wait()
            pltpu.make_async_copy(col_hbm.at[pl.ds(0, _CH)], cidx_v[b],
                                  isem[b]).wait()

        def issue_gather(b):
            pltpu.async_copy(table_sh.at[ridx_v[b]], a_v[b], gsem[b])
            pltpu.async_copy(table_sh.at[cidx_v[b]], b_v[b], gsem[b])

        def wait_gather(b):
            pltpu.make_async_copy(table_sh.at[ridx_v[b]], a_v[b],
                                  gsem[b]).wait()
            pltpu.make_async_copy(table_sh.at[cidx_v[b]], b_v[b],
                                  gsem[b]).wait()

        def issue_out(i, b):
            off = off_of(i)
            pltpu.async_copy(dist_v[b], dist_hbm.at[pl.ds(off, _CH)], osem[b])
            pltpu.async_copy(delta_v[b],
                             dflat_hbm.at[pl.ds(off * 4, _CH * 4)], osem[b])

        def wait_out(b):
            pltpu.make_async_copy(dist_v[b], dist_hbm.at[pl.ds(0, _CH)],
                                  osem[b]).wait()
            pltpu.make_async_copy(delta_v[b],
                                  dflat_hbm.at[pl.ds(0, _CH * 4)],
                                  osem[b]).wait()

        def compute(b):
            @pl.loop(0, ngrp)
            def _(g):
                gbase = g * 128
                pbase = g * 512
                for j in range(0, 128, _L):
                    r = lanes + (gbase + j)
                    k0 = jnp.zeros((_L,), jnp.int32)
                    k1 = k0 + 1
                    k2 = k0 + 2
                    ax = plsc.load_gather(a_v[b], [r, k0])
                    ay = plsc.load_gather(a_v[b], [r, k1])
                    az = plsc.load_gather(a_v[b], [r, k2])
                    bx = plsc.load_gather(b_v[b], [r, k0])
                    by = plsc.load_gather(b_v[b], [r, k1])
                    bz = plsc.load_gather(b_v[b], [r, k2])
                    mx = minimage(ax - bx)
                    my = minimage(ay - by)
                    mz = minimage(az - bz)
                    dx = mx * c00 + my * c10 + mz * c20
                    dy = mx * c01 + my * c11 + mz * c21
                    dz = mx * c02 + my * c12 + mz * c22
                    t = dx * dx + dy * dy + dz * dz + jnp.float32(1e-8)
                    i32v = plsc.bitcast(t, jnp.int32)
                    i32v = jnp.int32(0x5F3759DF) - (i32v >> 1)
                    y = plsc.bitcast(i32v, jnp.float32)
                    yh = t * half
                    y = y * (jnp.float32(1.5) - yh * y * y)
                    y = y * (jnp.float32(1.5) - yh * y * y)
                    y = y * (jnp.float32(1.5) - yh * y * y)
                    dist_v[b][pl.ds(gbase + j, _L)] = t * y
                    delta_v[b][pl.ds(pbase + j, _L)] = dx
                    delta_v[b][pl.ds(pbase + 128 + j, _L)] = dy
                    delta_v[b][pl.ds(pbase + 256 + j, _L)] = dz

        issue_idx(0, 0)
        issue_idx(1, 1)
        wait_idx(0)
        issue_gather(0)

        @pl.loop(0, maxt, step=2)
        def _(i0):
            for p in range(2):
                b = p
                nb = 1 - p
                i = i0 + p

                @pl.when(i < t_w)
                def _():
                    @pl.when(i + 1 < t_w)
                    def _():
                        wait_idx(nb)
                        issue_gather(nb)

                    wait_gather(b)

                    @pl.when(i + 2 < t_w)
                    def _():
                        issue_idx(i + 2, b)

                    @pl.when(i >= 2)
                    def _():
                        wait_out(b)

                    compute(b)
                    issue_out(i, b)

        wait_out(0)
        wait_out(1)

    return k(table, row, col, cell)


def kernel(pos, edge_index, cell, frac_coords):
    del pos
    n = frac_coords.shape[0]
    e = edge_index.shape[1]
    table = jnp.concatenate(
        [frac_coords.astype(jnp.float32),
         jnp.zeros((n, 5), jnp.float32)], axis=1)
    row, col = _split_edges_tc(edge_index)
    dist, dflat = _sc_periodic_distance(
        table, row, col, cell.astype(jnp.float32), e=e, n=n)
    delta = (dflat.reshape(e // 128, 4, 128)[:, :3, :]
             .transpose(0, 2, 1).reshape(e, 3))
    return dist, delta


# speedup vs baseline: 1.0025x; 1.0025x over previous
import dataclasses
import functools

import jax
import jax.numpy as jnp
from jax import lax
from jax.experimental import pallas as pl
from jax.experimental.pallas import tpu as pltpu
from jax.experimental.pallas import tpu_sc as plsc

_NC = 2
_NS = 16
_NW = _NC * _NS
_L = 16
_CH = 1280


def _compiler_params():
    cp = pltpu.CompilerParams()
    fields = pltpu.CompilerParams.__dataclass_fields__
    if "needs_layout_passes" in fields:
        cp = dataclasses.replace(cp, needs_layout_passes=False)
    if "use_tc_tiling_on_sc" in fields:
        cp = dataclasses.replace(cp, use_tc_tiling_on_sc=False)
    return cp


def _split_edges_tc(edge_index):
    e = edge_index.shape[1]
    blk = 1
    for cand in (256000, 51200, 10240, 2048, 1024):
        if e % cand == 0:
            blk = cand
            break
    grid = e // blk

    def body(x_ref, r_ref, c_ref):
        r_ref[...] = x_ref[0, :]
        c_ref[...] = x_ref[1, :]

    return pl.pallas_call(
        body,
        grid=(grid,),
        in_specs=[pl.BlockSpec((2, blk), lambda i: (0, i))],
        out_specs=(
            pl.BlockSpec((blk,), lambda i: (i,)),
            pl.BlockSpec((blk,), lambda i: (i,)),
        ),
        out_shape=(
            jax.ShapeDtypeStruct((e,), jnp.int32),
            jax.ShapeDtypeStruct((e,), jnp.int32),
        ),
    )(edge_index)


@functools.partial(jax.jit, static_argnames=("e", "n"))
def _sc_periodic_distance(table, row, col, cell, *, e, n):
    del n
    assert e % _CH == 0 and e % 128 == 0
    n_chunks = e // _CH
    nfull, rem = divmod(n_chunks, _NW)
    maxt = nfull + (1 if rem else 0)
    maxt += maxt % 2
    ngrp = _CH // 128

    mesh = plsc.VectorSubcoreMesh(core_axis_name="c", subcore_axis_name="s")

    @functools.partial(
        pl.kernel,
        compiler_params=_compiler_params(),
        out_type=(
            jax.ShapeDtypeStruct((e,), jnp.float32),
            jax.ShapeDtypeStruct((4 * e,), jnp.float32),
        ),
        mesh=mesh,
        scratch_types=[
            [pltpu.VMEM((_CH,), jnp.int32) for _ in range(2)],
            [pltpu.VMEM((_CH,), jnp.int32) for _ in range(2)],
            [pltpu.VMEM((_CH, 8), jnp.float32) for _ in range(2)],
            [pltpu.VMEM((_CH, 8), jnp.float32) for _ in range(2)],
            [pltpu.VMEM((_CH,), jnp.float32) for _ in range(2)],
            [pltpu.VMEM((4 * _CH,), jnp.float32) for _ in range(2)],
            pltpu.VMEM((3, 3), jnp.float32),
            pltpu.VMEM_SHARED((100000, 8), jnp.float32),
            [pltpu.SemaphoreType.DMA for _ in range(2)],
            [pltpu.SemaphoreType.DMA for _ in range(2)],
            [pltpu.SemaphoreType.DMA for _ in range(2)],
        ],
    )
    def k(table_hbm, row_hbm, col_hbm, cell_hbm, dist_hbm, dflat_hbm,
          ridx_v, cidx_v, a_v, b_v, dist_v, delta_v, cell_v, table_sh,
          isem, gsem, osem):
        wid = lax.axis_index("c") * _NS + lax.axis_index("s")
        t_w = nfull + jnp.where(wid < rem, 1, 0)
        pltpu.sync_copy(cell_hbm, cell_v)

        @pl.when(lax.axis_index("s") == 0)
        def _():
            pltpu.sync_copy(table_hbm, table_sh)

        plsc.subcore_barrier()

        lanes = lax.iota(jnp.int32, _L)
        crow = jnp.where(lanes < 9, lanes, 0) // 3
        ccol = jnp.where(lanes < 9, lanes, 0) % 3
        cv = plsc.load_gather(cell_v, [crow, ccol])
        c00 = cv[0]
        c01 = cv[1]
        c02 = cv[2]
        c10 = cv[3]
        c11 = cv[4]
        c12 = cv[5]
        c20 = cv[6]
        c21 = cv[7]
        c22 = cv[8]

        half = jnp.float32(0.5)
        one = jnp.float32(1.0)
        zeros = jnp.zeros((_L,), jnp.float32)

        for b in range(2):
            for g in range(ngrp):
                for j in range(0, 128, _L):
                    delta_v[b][pl.ds(g * 512 + 384 + j, _L)] = zeros

        def minimage(d):
            d = jnp.where(d > half, d - one, d)
            return jnp.where(d < -half, d + one, d)

        def off_of(i):
            return (wid + _NW * i) * _CH

        def issue_idx(i, b):
            off = off_of(i)
            pltpu.async_copy(row_hbm.at[pl.ds(off, _CH)], ridx_v[b], isem[b])
            pltpu.async_copy(col_hbm.at[pl.ds(off, _CH)], cidx_v[b], isem[b])

        def wait_idx(b):
            pltpu.make_async_copy(row_hbm.at[pl.ds(0, _CH)], ridx_v[b],
                                  isem[b]).wait()
            pltpu.make_async_copy(col_hbm.at[pl.ds(0, _CH)], cidx_v[b],
                                  isem[b]).wait()

        def issue_gather(b):
            pltpu.async_copy(table_sh.at[ridx_v[b]], a_v[b], gsem[b])
            pltpu.async_copy(table_sh.at[cidx_v[b]], b_v[b], gsem[b])

        def wait_gather(b):
            pltpu.make_async_copy(table_sh.at[ridx_v[b]], a_v[b],
                                  gsem[b]).wait()
            pltpu.make_async_copy(table_sh.at[cidx_v[b]], b_v[b],
                                  gsem[b]).wait()

        def issue_out(i, b):
            off = off_of(i)
            pltpu.async_copy(dist_v[b], dist_hbm.at[pl.ds(off, _CH)], osem[b])
            pltpu.async_copy(delta_v[b],
                             dflat_hbm.at[pl.ds(off * 4, _CH * 4)], osem[b])

        def wait_out(b):
            pltpu.make_async_copy(dist_v[b], dist_hbm.at[pl.ds(0, _CH)],
                                  osem[b]).wait()
            pltpu.make_async_copy(delta_v[b],
                                  dflat_hbm.at[pl.ds(0, _CH * 4)],
                                  osem[b]).wait()

        def compute(b):
            @pl.loop(0, ngrp)
            def _(g):
                gbase = g * 128
                pbase = g * 512
                for j in range(0, 128, _L):
                    r = lanes + (gbase + j)
                    k0 = jnp.zeros((_L,), jnp.int32)
                    k1 = k0 + 1
                    k2 = k0 + 2
                    ax = plsc.load_gather(a_v[b], [r, k0])
                    ay = plsc.load_gather(a_v[b], [r, k1])
                    az = plsc.load_gather(a_v[b], [r, k2])
                    bx = plsc.load_gather(b_v[b], [r, k0])
                    by = plsc.load_gather(b_v[b], [r, k1])
                    bz = plsc.load_gather(b_v[b], [r, k2])
                    mx = minimage(ax - bx)
                    my = minimage(ay - by)
                    mz = minimage(az - bz)
                    dx = mx * c00 + my * c10 + mz * c20
                    dy = mx * c01 + my * c11 + mz * c21
                    dz = mx * c02 + my * c12 + mz * c22
                    t = dx * dx + dy * dy + dz * dz + jnp.float32(1e-8)
                    i32v = plsc.bitcast(t, jnp.int32)
                    i32v = jnp.int32(0x5F3759DF) - (i32v >> 1)
                    y = plsc.bitcast(i32v, jnp.float32)
                    yh = t * half
                    y = y * (jnp.float32(1.5) - yh * y * y)
                    y = y * (jnp.float32(1.5) - yh * y * y)
                    y = y * (jnp.float32(1.5) - yh * y * y)
                    dist_v[b][pl.ds(gbase + j, _L)] = t * y
                    delta_v[b][pl.ds(pbase + j, _L)] = dx
                    delta_v[b][pl.ds(pbase + 128 + j, _L)] = dy
                    delta_v[b][pl.ds(pbase + 256 + j, _L)] = dz

        issue_idx(0, 0)
        issue_idx(1, 1)
        wait_idx(0)
        issue_gather(0)

        @pl.loop(0, maxt, step=2)
        def _(i0):
            for p in range(2):
                b = p
                nb = 1 - p
                i = i0 + p

                @pl.when(i < t_w)
                def _():
                    @pl.when(i + 1 < t_w)
                    def _():
                        wait_idx(nb)
                        issue_gather(nb)

                    wait_gather(b)

                    @pl.when(i + 2 < t_w)
                    def _():
                        issue_idx(i + 2, b)

                    @pl.when(i >= 2)
                    def _():
                        wait_out(b)

                    compute(b)
                    issue_out(i, b)

        wait_out(0)
        wait_out(1)

    return k(table, row, col, cell)


def kernel(pos, edge_index, cell, frac_coords):
    del pos
    n = frac_coords.shape[0]
    e = edge_index.shape[1]
    table = jnp.concatenate(
        [frac_coords.astype(jnp.float32),
         jnp.zeros((n, 5), jnp.float32)], axis=1)
    row, col = _split_edges_tc(edge_index)
    dist, dflat = _sc_periodic_distance(
        table, row, col, cell.astype(jnp.float32), e=e, n=n)
    delta = (dflat.reshape(e // 128, 4, 128)[:, :3, :]
             .transpose(0, 2, 1).reshape(e, 3))
    return dist, delta


# speedup vs baseline: 1.0947x; 1.0920x over previous
import dataclasses
import functools

import jax
import jax.numpy as jnp
from jax import lax
from jax.experimental import pallas as pl
from jax.experimental.pallas import tpu as pltpu
from jax.experimental.pallas import tpu_sc as plsc

_NC = 2
_NS = 16
_NW = _NC * _NS
_L = 16
_CH = 1280


def _compiler_params():
    cp = pltpu.CompilerParams()
    fields = pltpu.CompilerParams.__dataclass_fields__
    if "needs_layout_passes" in fields:
        cp = dataclasses.replace(cp, needs_layout_passes=False)
    if "use_tc_tiling_on_sc" in fields:
        cp = dataclasses.replace(cp, use_tc_tiling_on_sc=False)
    return cp


def _split_edges_tc(edge_index):
    e = edge_index.shape[1]
    blk = 1
    for cand in (256000, 51200, 10240, 2048, 1024):
        if e % cand == 0:
            blk = cand
            break
    grid = e // blk

    def body(x_ref, r_ref, c_ref):
        r_ref[...] = x_ref[0, :]
        c_ref[...] = x_ref[1, :]

    return pl.pallas_call(
        body,
        grid=(grid,),
        in_specs=[pl.BlockSpec((2, blk), lambda i: (0, i))],
        out_specs=(
            pl.BlockSpec((blk,), lambda i: (i,)),
            pl.BlockSpec((blk,), lambda i: (i,)),
        ),
        out_shape=(
            jax.ShapeDtypeStruct((e,), jnp.int32),
            jax.ShapeDtypeStruct((e,), jnp.int32),
        ),
    )(edge_index)


@functools.partial(jax.jit, static_argnames=("e", "n"))
def _sc_periodic_distance(table, row, col, cell, *, e, n):
    del n
    assert e % _CH == 0 and e % 128 == 0
    n_chunks = e // _CH
    nfull, rem = divmod(n_chunks, _NW)
    maxt = nfull + (1 if rem else 0)
    maxt += maxt % 2
    ngrp = _CH // 128

    mesh = plsc.VectorSubcoreMesh(core_axis_name="c", subcore_axis_name="s")

    @functools.partial(
        pl.kernel,
        compiler_params=_compiler_params(),
        out_type=(
            jax.ShapeDtypeStruct((e,), jnp.float32),
            jax.ShapeDtypeStruct((4 * e,), jnp.float32),
        ),
        mesh=mesh,
        scratch_types=[
            [pltpu.VMEM((_CH,), jnp.int32) for _ in range(2)],
            [pltpu.VMEM((_CH,), jnp.int32) for _ in range(2)],
            [pltpu.VMEM((_CH, 8), jnp.float32) for _ in range(2)],
            [pltpu.VMEM((_CH, 8), jnp.float32) for _ in range(2)],
            [pltpu.VMEM((_CH,), jnp.float32) for _ in range(2)],
            [pltpu.VMEM((4 * _CH,), jnp.float32) for _ in range(2)],
            pltpu.VMEM((3, 3), jnp.float32),
            pltpu.VMEM_SHARED((100000, 8), jnp.float32),
            [pltpu.SemaphoreType.DMA for _ in range(2)],
            [pltpu.SemaphoreType.DMA for _ in range(2)],
            [pltpu.SemaphoreType.DMA for _ in range(2)],
        ],
    )
    def k(table_hbm, row_hbm, col_hbm, cell_hbm, dist_hbm, dflat_hbm,
          ridx_v, cidx_v, a_v, b_v, dist_v, delta_v, cell_v, table_sh,
          isem, gsem, osem):
        wid = lax.axis_index("c") * _NS + lax.axis_index("s")
        t_w = nfull + jnp.where(wid < rem, 1, 0)
        pltpu.sync_copy(cell_hbm, cell_v)

        @pl.when(lax.axis_index("s") == 0)
        def _():
            pltpu.sync_copy(table_hbm, table_sh)

        plsc.subcore_barrier()

        lanes = lax.iota(jnp.int32, _L)
        crow = jnp.where(lanes < 9, lanes, 0) // 3
        ccol = jnp.where(lanes < 9, lanes, 0) % 3
        cv = plsc.load_gather(cell_v, [crow, ccol])
        c00 = cv[0]
        c01 = cv[1]
        c02 = cv[2]
        c10 = cv[3]
        c11 = cv[4]
        c12 = cv[5]
        c20 = cv[6]
        c21 = cv[7]
        c22 = cv[8]

        half = jnp.float32(0.5)
        one = jnp.float32(1.0)
        zeros = jnp.zeros((_L,), jnp.float32)

        for b in range(2):
            for g in range(ngrp):
                for j in range(0, 128, _L):
                    delta_v[b][pl.ds(g * 512 + 384 + j, _L)] = zeros

        def minimage(d):
            d = jnp.where(d > half, d - one, d)
            return jnp.where(d < -half, d + one, d)

        def off_of(i):
            return (wid + _NW * i) * _CH

        def issue_idx(i, b):
            off = off_of(i)
            pltpu.async_copy(row_hbm.at[pl.ds(off, _CH)], ridx_v[b], isem[b])
            pltpu.async_copy(col_hbm.at[pl.ds(off, _CH)], cidx_v[b], isem[b])

        def wait_idx(b):
            pltpu.make_async_copy(row_hbm.at[pl.ds(0, _CH)], ridx_v[b],
                                  isem[b]).wait()
            pltpu.make_async_copy(col_hbm.at[pl.ds(0, _CH)], cidx_v[b],
                                  isem[b]).wait()

        def issue_gather(b):
            pltpu.async_copy(table_sh.at[ridx_v[b]], a_v[b], gsem[b])
            pltpu.async_copy(table_sh.at[cidx_v[b]], b_v[b], gsem[b])

        def wait_gather(b):
            pltpu.make_async_copy(table_sh.at[ridx_v[b]], a_v[b],
                                  gsem[b]).wait()
            pltpu.make_async_copy(table_sh.at[cidx_v[b]], b_v[b],
                                  gsem[b]).wait()

        def issue_out(i, b):
            off = off_of(i)
            pltpu.async_copy(dist_v[b], dist_hbm.at[pl.ds(off, _CH)], osem[b])
            pltpu.async_copy(delta_v[b],
                             dflat_hbm.at[pl.ds(off * 4, _CH * 4)], osem[b])

        def wait_out(b):
            pltpu.make_async_copy(dist_v[b], dist_hbm.at[pl.ds(0, _CH)],
                                  osem[b]).wait()
            pltpu.make_async_copy(delta_v[b],
                                  dflat_hbm.at[pl.ds(0, _CH * 4)],
                                  osem[b]).wait()

        def compute(b):
            @pl.loop(0, ngrp)
            def _(g):
                gbase = g * 128
                pbase = g * 512
                for j in range(0, 128, _L):
                    r = lanes + (gbase + j)
                    k0 = jnp.zeros((_L,), jnp.int32)
                    k1 = k0 + 1
                    k2 = k0 + 2
                    ax = plsc.load_gather(a_v[b], [r, k0])
                    ay = plsc.load_gather(a_v[b], [r, k1])
                    az = plsc.load_gather(a_v[b], [r, k2])
                    bx = plsc.load_gather(b_v[b], [r, k0])
                    by = plsc.load_gather(b_v[b], [r, k1])
                    bz = plsc.load_gather(b_v[b], [r, k2])
                    mx = minimage(ax - bx)
                    my = minimage(ay - by)
                    mz = minimage(az - bz)
                    dx = mx * c00 + my * c10 + mz * c20
                    dy = mx * c01 + my * c11 + mz * c21
                    dz = mx * c02 + my * c12 + mz * c22
                    t = dx * dx + dy * dy + dz * dz + jnp.float32(1e-8)
                    i32v = plsc.bitcast(t, jnp.int32)
                    i32v = jnp.int32(0x5F3759DF) - (i32v >> 1)
                    y = plsc.bitcast(i32v, jnp.float32)
                    yh = t * half
                    y = y * (jnp.float32(1.5) - yh * y * y)
                    y = y * (jnp.float32(1.5) - yh * y * y)
                    y = y * (jnp.float32(1.5) - yh * y * y)
                    dist_v[b][pl.ds(gbase + j, _L)] = t * y
                    delta_v[b][pl.ds(pbase + j, _L)] = dx
                    delta_v[b][pl.ds(pbase + 128 + j, _L)] = dy
                    delta_v[b][pl.ds(pbase + 256 + j, _L)] = dz

        issue_idx(0, 0)
        issue_idx(1, 1)
        wait_idx(0)
        issue_gather(0)

        @pl.loop(0, maxt, step=2)
        def _(i0):
            for p in range(2):
                b = p
                nb = 1 - p
                i = i0 + p

                @pl.when(i < t_w)
                def _():
                    @pl.when(i + 1 < t_w)
                    def _():
                        wait_idx(nb)
                        issue_gather(nb)

                    wait_gather(b)

                    @pl.when(i + 2 < t_w)
                    def _():
                        issue_idx(i + 2, b)

                    @pl.when(i >= 2)
                    def _():
                        wait_out(b)

                    compute(b)
                    issue_out(i, b)

        wait_out(0)
        wait_out(1)

    return k(table, row, col, cell)


def kernel(pos, edge_index, cell, frac_coords):
    del pos
    n = frac_coords.shape[0]
    e = edge_index.shape[1]
    table = jnp.concatenate(
        [frac_coords.astype(jnp.float32),
         jnp.zeros((n, 5), jnp.float32)], axis=1)
    row, col = _split_edges_tc(edge_index)
    dist, dflat = _sc_periodic_distance(
        table, row, col, cell.astype(jnp.float32), e=e, n=n)
    delta = (dflat.reshape(e // 128, 4, 128)
             .transpose(0, 2, 1).reshape(e, 4)[:, :3])
    return dist, delta
